# baseline (speedup 1.0000x reference)
import jax
import jax.numpy as jnp
from jax.experimental import pallas as pl
from jax.experimental.pallas import tpu as pltpu

N_WIN = 8
NUM_HEADS = 8
TOPK = 4
NEG = -1000000000.0


def _fused_kernel(x_ref, wqkv_ref, bqkv_ref, wproj_ref, bproj_ref, out_ref):
    N, C = x_ref.shape
    head_dim = C // NUM_HEADS
    win = N // N_WIN
    scale = head_dim ** (-0.5)

    x = x_ref[...]
    wqkv = wqkv_ref[...]
    qkv = jax.lax.dot_general(x, wqkv, (((1,), (1,)), ((), ())),
                              preferred_element_type=jnp.float32)
    qkv = qkv + bqkv_ref[...]
    spikes = (qkv / 2.0 - 1.0 >= 0.0).astype(jnp.float32)

    row_w = jax.lax.broadcasted_iota(jnp.int32, (N, N_WIN), 0) // win
    col_w = jax.lax.broadcasted_iota(jnp.int32, (N, N_WIN), 1)
    Rwin = (row_w == col_w).astype(jnp.float32)
    Awin = (row_w == col_w).astype(jnp.float32)

    ones_hd = jnp.ones((1, head_dim), dtype=jnp.float32)
    lane8 = jax.lax.broadcasted_iota(jnp.int32, (N_WIN, N_WIN), 1)

    y = jnp.zeros((N, C), dtype=jnp.float32)
    wproj = wproj_ref[...]
    for h in range(NUM_HEADS):
        qh = spikes[:, h * head_dim:(h + 1) * head_dim]
        kh = spikes[:, C + h * head_dim:C + (h + 1) * head_dim]
        vh = spikes[:, 2 * C + h * head_dim:2 * C + (h + 1) * head_dim]

        q_reg = jax.lax.dot_general(Awin, qh, (((0,), (0,)), ((), ())),
                                    preferred_element_type=jnp.float32)
        k_reg = jax.lax.dot_general(Awin, kh, (((0,), (0,)), ((), ())),
                                    preferred_element_type=jnp.float32)

        act = jax.lax.dot_general(ones_hd, k_reg, (((1,), (1,)), ((), ())),
                                  preferred_element_type=jnp.float32)
        act_mask = (act > 1e-05).astype(jnp.float32)

        sim = jax.lax.dot_general(q_reg, k_reg, (((1,), (1,)), ((), ())),
                                  preferred_element_type=jnp.float32)
        simm = sim * scale + (1.0 - act_mask) * NEG

        cnt = jnp.zeros((N_WIN, N_WIN), dtype=jnp.float32)
        for jp in range(N_WIN):
            colv = simm[:, jp:jp + 1]
            gt = (colv > simm).astype(jnp.float32)
            eq = ((colv == simm) & (jp < lane8)).astype(jnp.float32)
            cnt = cnt + gt + eq
        sel = (cnt < float(TOPK)).astype(jnp.float32)
        negrow = (1.0 - sel) * NEG
        neg_keys = jax.lax.dot_general(negrow, Rwin, (((1,), (1,)), ((), ())),
                                       preferred_element_type=jnp.float32)

        lint = jax.lax.dot_general(qh.astype(jnp.bfloat16), kh.astype(jnp.bfloat16),
                                   (((1,), (1,)), ((), ())),
                                   preferred_element_type=jnp.float32)
        pieces = []
        for w in range(N_WIN):
            pieces.append(lint[w * win:(w + 1) * win] * scale + neg_keys[w:w + 1, :])
        logits = jnp.concatenate(pieces, axis=0)

        m = jnp.max(logits, axis=1, keepdims=True)
        p = jnp.exp(logits - m)
        s = jnp.sum(p, axis=1, keepdims=True)
        attn = p / s

        out_h = jax.lax.dot_general(attn, vh, (((1,), (0,)), ((), ())),
                                    preferred_element_type=jnp.float32)
        wp_h = wproj[:, h * head_dim:(h + 1) * head_dim]
        y = y + jax.lax.dot_general(out_h, wp_h, (((1,), (1,)), ((), ())),
                                    preferred_element_type=jnp.float32)

    y = y + bproj_ref[...]
    out_ref[...] = (y / 2.0 - 1.0 >= 0.0).astype(jnp.float32)


def kernel(x, Wqkv, bqkv, Wproj, bproj):
    T, B, N, C = x.shape
    xf = x.reshape(T * B, N, C).reshape(T * B * N, C)
    out = pl.pallas_call(
        _fused_kernel,
        grid=(T * B,),
        in_specs=[
            pl.BlockSpec((N, C), lambda i: (i, 0)),
            pl.BlockSpec((3 * C, C), lambda i: (0, 0)),
            pl.BlockSpec((1, 3 * C), lambda i: (0, 0)),
            pl.BlockSpec((C, C), lambda i: (0, 0)),
            pl.BlockSpec((1, C), lambda i: (0, 0)),
        ],
        out_specs=pl.BlockSpec((N, C), lambda i: (i, 0)),
        out_shape=jax.ShapeDtypeStruct((T * B * N, C), jnp.float32),
        compiler_params=pltpu.CompilerParams(
            dimension_semantics=("parallel",),
        ),
    )(xf, Wqkv, bqkv.reshape(1, 3 * C), Wproj, bproj.reshape(1, C))
    return out.reshape(T, B, N, C)


# speedup vs baseline: 1.0521x; 1.0521x over previous
import jax
import jax.numpy as jnp
from jax.experimental import pallas as pl
from jax.experimental.pallas import tpu as pltpu

N_WIN = 8
NUM_HEADS = 8
TOPK = 4
NEG = -1000000000.0


def _fused_kernel(x_ref, wqkv_ref, bqkv_ref, wproj_ref, bproj_ref, out_ref):
    N, C = x_ref.shape
    head_dim = C // NUM_HEADS
    win = N // N_WIN
    scale = head_dim ** (-0.5)

    x = x_ref[...]
    wqkv = wqkv_ref[...]
    qkv = jax.lax.dot_general(x, wqkv, (((1,), (1,)), ((), ())),
                              preferred_element_type=jnp.float32)
    qkv = qkv + bqkv_ref[...]
    spikes = (qkv / 2.0 - 1.0 >= 0.0).astype(jnp.bfloat16)

    row_w = jax.lax.broadcasted_iota(jnp.int32, (N, N_WIN), 0) // win
    col_w = jax.lax.broadcasted_iota(jnp.int32, (N, N_WIN), 1)
    Rwin = (row_w == col_w).astype(jnp.bfloat16)

    ones_hd = jnp.ones((1, head_dim), dtype=jnp.float32)
    ones_col = jnp.ones((N, 1), dtype=jnp.bfloat16)
    lane8 = jax.lax.broadcasted_iota(jnp.int32, (N_WIN, N_WIN), 1)
    eye8 = (jax.lax.broadcasted_iota(jnp.int32, (N_WIN, N_WIN), 0)
            == lane8).astype(jnp.float32)

    y = jnp.zeros((N, C), dtype=jnp.float32)
    wproj = wproj_ref[...]
    for h in range(NUM_HEADS):
        qh = spikes[:, h * head_dim:(h + 1) * head_dim]
        kh = spikes[:, C + h * head_dim:C + (h + 1) * head_dim]
        vh = spikes[:, 2 * C + h * head_dim:2 * C + (h + 1) * head_dim]

        q_reg = jax.lax.dot_general(Rwin, qh, (((0,), (0,)), ((), ())),
                                    preferred_element_type=jnp.float32)
        k_reg = jax.lax.dot_general(Rwin, kh, (((0,), (0,)), ((), ())),
                                    preferred_element_type=jnp.float32)

        act = jax.lax.dot_general(ones_hd, k_reg, (((1,), (1,)), ((), ())),
                                  preferred_element_type=jnp.float32)
        act_mask = (act > 1e-05).astype(jnp.float32)

        sim = jax.lax.dot_general(q_reg, k_reg, (((1,), (1,)), ((), ())),
                                  preferred_element_type=jnp.float32)
        simm = sim * scale + (1.0 - act_mask) * NEG

        cnt = jnp.zeros((N_WIN, N_WIN), dtype=jnp.float32)
        for jp in range(N_WIN):
            colv = simm[:, jp:jp + 1]
            gt = (colv > simm).astype(jnp.float32)
            eq = ((colv == simm) & (jp < lane8)).astype(jnp.float32)
            cnt = cnt + gt + eq
        sel = (cnt < float(TOPK)).astype(jnp.float32)
        negrow = (1.0 - sel) * NEG
        negrowT = jax.lax.dot_general(negrow, eye8, (((0,), (0,)), ((), ())),
                                      preferred_element_type=jnp.float32)
        negT = jax.lax.dot_general(Rwin.astype(jnp.float32), negrowT,
                                   (((1,), (0,)), ((), ())),
                                   preferred_element_type=jnp.float32)

        qfull = jnp.concatenate([qh, Rwin], axis=1)
        kfull = jnp.concatenate([kh, negT.astype(jnp.bfloat16)], axis=1)
        lint = jax.lax.dot_general(qfull, kfull, (((1,), (1,)), ((), ())),
                                   preferred_element_type=jnp.float32)

        p = jnp.exp(lint * scale).astype(jnp.bfloat16)
        vfull = jnp.concatenate([vh, ones_col], axis=1)
        num = jax.lax.dot_general(p, vfull, (((1,), (0,)), ((), ())),
                                  preferred_element_type=jnp.float32)
        r = 1.0 / num[:, head_dim:head_dim + 1]
        out_h = num[:, :head_dim] * r
        wp_h = wproj[:, h * head_dim:(h + 1) * head_dim]
        y = y + jax.lax.dot_general(out_h, wp_h, (((1,), (1,)), ((), ())),
                                    preferred_element_type=jnp.float32)

    y = y + bproj_ref[...]
    out_ref[...] = (y / 2.0 - 1.0 >= 0.0).astype(jnp.float32)


def kernel(x, Wqkv, bqkv, Wproj, bproj):
    T, B, N, C = x.shape
    xf = x.reshape(T * B, N, C).reshape(T * B * N, C)
    out = pl.pallas_call(
        _fused_kernel,
        grid=(T * B,),
        in_specs=[
            pl.BlockSpec((N, C), lambda i: (i, 0)),
            pl.BlockSpec((3 * C, C), lambda i: (0, 0)),
            pl.BlockSpec((1, 3 * C), lambda i: (0, 0)),
            pl.BlockSpec((C, C), lambda i: (0, 0)),
            pl.BlockSpec((1, C), lambda i: (0, 0)),
        ],
        out_specs=pl.BlockSpec((N, C), lambda i: (i, 0)),
        out_shape=jax.ShapeDtypeStruct((T * B * N, C), jnp.float32),
        compiler_params=pltpu.CompilerParams(
            dimension_semantics=("parallel",),
        ),
    )(xf, Wqkv, bqkv.reshape(1, 3 * C), Wproj, bproj.reshape(1, C))
    return out.reshape(T, B, N, C)


# speedup vs baseline: 1.4641x; 1.3916x over previous
import jax
import jax.numpy as jnp
from jax.experimental import pallas as pl
from jax.experimental.pallas import tpu as pltpu

N_WIN = 8
NUM_HEADS = 8
TOPK = 4
NEG = -1000000000.0


def _fused_kernel(x_ref, wqkv_ref, bqkv_ref, wproj_ref, bproj_ref, out_ref):
    N, C = x_ref.shape
    head_dim = C // NUM_HEADS
    win = N // N_WIN
    scale = head_dim ** (-0.5)

    x = x_ref[...]
    wqkv = wqkv_ref[...]
    qkv = jax.lax.dot_general(x, wqkv, (((1,), (1,)), ((), ())),
                              preferred_element_type=jnp.float32)
    qkv = qkv + bqkv_ref[...]
    spikes = (qkv / 2.0 - 1.0 >= 0.0).astype(jnp.bfloat16)

    row_w = jax.lax.broadcasted_iota(jnp.int32, (N, N_WIN), 0) // win
    col_w = jax.lax.broadcasted_iota(jnp.int32, (N, N_WIN), 1)
    Rwin = (row_w == col_w).astype(jnp.bfloat16)
    Rwin32 = (row_w == col_w).astype(jnp.float32)

    HW = NUM_HEADS * N_WIN
    ones_col = jnp.ones((N, 1), dtype=jnp.bfloat16)
    lane8_64 = jax.lax.broadcasted_iota(jnp.int32, (HW, N_WIN), 1)
    eye64 = (jax.lax.broadcasted_iota(jnp.int32, (HW, HW), 0)
             == jax.lax.broadcasted_iota(jnp.int32, (HW, HW), 1)).astype(jnp.float32)
    eye8 = (jax.lax.broadcasted_iota(jnp.int32, (N_WIN, N_WIN), 0)
            == jax.lax.broadcasted_iota(jnp.int32, (N_WIN, N_WIN), 1)).astype(jnp.float32)
    hcol = jax.lax.broadcasted_iota(jnp.int32, (C, NUM_HEADS), 0) // head_dim
    hsel = (hcol == jax.lax.broadcasted_iota(jnp.int32, (C, NUM_HEADS), 1)
            ).astype(jnp.float32)

    reg_all = jax.lax.dot_general(Rwin, spikes, (((0,), (0,)), ((), ())),
                                  preferred_element_type=jnp.float32)
    kreg_all = reg_all[:, C:2 * C]
    act_mat = jax.lax.dot_general(kreg_all, hsel, (((1,), (0,)), ((), ())),
                                  preferred_element_type=jnp.float32)
    act_rows = jax.lax.dot_general(act_mat, eye8, (((0,), (0,)), ((), ())),
                                   preferred_element_type=jnp.float32)

    simm_parts = []
    for h in range(NUM_HEADS):
        q_reg = reg_all[:, h * head_dim:(h + 1) * head_dim]
        k_reg = reg_all[:, C + h * head_dim:C + (h + 1) * head_dim]
        sim = jax.lax.dot_general(q_reg, k_reg, (((1,), (1,)), ((), ())),
                                  preferred_element_type=jnp.float32)
        actm = (act_rows[h:h + 1, :] > 1e-05).astype(jnp.float32)
        simm_parts.append(sim * scale + (1.0 - actm) * NEG)
    simm_all = jnp.concatenate(simm_parts, axis=0)

    cnt = jnp.zeros((HW, N_WIN), dtype=jnp.float32)
    for jp in range(N_WIN):
        colv = simm_all[:, jp:jp + 1]
        gt = (colv > simm_all).astype(jnp.float32)
        eq = ((colv == simm_all) & (jp < lane8_64)).astype(jnp.float32)
        cnt = cnt + gt + eq
    sel_all = (cnt < float(TOPK)).astype(jnp.float32)
    negrow_all = (1.0 - sel_all) * NEG
    negrow_allT = jax.lax.dot_general(negrow_all, eye64, (((0,), (0,)), ((), ())),
                                      preferred_element_type=jnp.float32)
    negT_all = jax.lax.dot_general(Rwin32, negrow_allT, (((1,), (0,)), ((), ())),
                                   preferred_element_type=jnp.float32)
    negT_all = negT_all.astype(jnp.bfloat16)

    QSC = float(scale * 1.4426950408889634)
    y = jnp.zeros((N, C), dtype=jnp.float32)
    wproj = wproj_ref[...]
    for h in range(NUM_HEADS):
        qh = spikes[:, h * head_dim:(h + 1) * head_dim]
        kh = spikes[:, C + h * head_dim:C + (h + 1) * head_dim]
        vh = spikes[:, 2 * C + h * head_dim:2 * C + (h + 1) * head_dim]

        qfull = jnp.concatenate([qh * QSC, Rwin], axis=1)
        kfull = jnp.concatenate([kh, negT_all[:, h * N_WIN:(h + 1) * N_WIN]],
                                axis=1)
        lint = jax.lax.dot_general(qfull, kfull, (((1,), (1,)), ((), ())),
                                   preferred_element_type=jnp.float32)

        p = jnp.exp2(lint).astype(jnp.bfloat16)
        vfull = jnp.concatenate([vh, ones_col], axis=1)
        num = jax.lax.dot_general(p, vfull, (((1,), (0,)), ((), ())),
                                  preferred_element_type=jnp.float32)
        r = 1.0 / num[:, head_dim:head_dim + 1]
        out_h = num[:, :head_dim] * r
        wp_h = wproj[:, h * head_dim:(h + 1) * head_dim]
        y = y + jax.lax.dot_general(out_h, wp_h, (((1,), (1,)), ((), ())),
                                    preferred_element_type=jnp.float32)

    y = y + bproj_ref[...]
    out_ref[...] = (y / 2.0 - 1.0 >= 0.0).astype(jnp.float32)


def kernel(x, Wqkv, bqkv, Wproj, bproj):
    T, B, N, C = x.shape
    xf = x.reshape(T * B, N, C).reshape(T * B * N, C)
    out = pl.pallas_call(
        _fused_kernel,
        grid=(T * B,),
        in_specs=[
            pl.BlockSpec((N, C), lambda i: (i, 0)),
            pl.BlockSpec((3 * C, C), lambda i: (0, 0)),
            pl.BlockSpec((1, 3 * C), lambda i: (0, 0)),
            pl.BlockSpec((C, C), lambda i: (0, 0)),
            pl.BlockSpec((1, C), lambda i: (0, 0)),
        ],
        out_specs=pl.BlockSpec((N, C), lambda i: (i, 0)),
        out_shape=jax.ShapeDtypeStruct((T * B * N, C), jnp.float32),
        compiler_params=pltpu.CompilerParams(
            dimension_semantics=("parallel",),
        ),
    )(xf, Wqkv, bqkv.reshape(1, 3 * C), Wproj, bproj.reshape(1, C))
    return out.reshape(T, B, N, C)
